# baseline (speedup 1.0000x reference)
import functools

import jax
import jax.numpy as jnp
from jax import lax
from jax.experimental import pallas as pl
from jax.experimental.pallas import tpu as pltpu
from jax.experimental.pallas import tpu_sc as plsc

N = 10000
D = 128
E = 320000
NHEAD = 2
HDIM = D // NHEAD
DFF = 2048
HID = 256

_RB1 = 400


def _prep_body(beta_ref, x_ref, xn_ref, xnb_ref, selfnum_ref, selfw_ref,
               norm_ref):
    x = x_ref[...]
    beta = beta_ref[0]
    nsq = jnp.sum(x * x, axis=1, keepdims=True)
    nrm = jnp.maximum(jnp.sqrt(nsq), 1e-12)
    xn = x / nrm
    xn_ref[...] = xn
    xnb_ref[...] = xn * beta
    sw = jnp.exp(beta * jnp.sum(xn * xn, axis=1, keepdims=True))
    selfnum_ref[...] = x * sw
    selfw_ref[...] = sw
    norm_ref[...] = nrm


def _prep(x, beta):
    grid = N // _RB1
    return pl.pallas_call(
        _prep_body,
        grid=grid,
        in_specs=[
            pl.BlockSpec(memory_space=pltpu.SMEM),
            pl.BlockSpec((_RB1, D), lambda i: (i, 0)),
        ],
        out_specs=[
            pl.BlockSpec((_RB1, D), lambda i: (i, 0)),
            pl.BlockSpec((_RB1, D), lambda i: (i, 0)),
            pl.BlockSpec((_RB1, D), lambda i: (i, 0)),
            pl.BlockSpec((_RB1, 1), lambda i: (i, 0)),
            pl.BlockSpec((_RB1, 1), lambda i: (i, 0)),
        ],
        out_shape=[
            jax.ShapeDtypeStruct((N, D), jnp.float32),
            jax.ShapeDtypeStruct((N, D), jnp.float32),
            jax.ShapeDtypeStruct((N, D), jnp.float32),
            jax.ShapeDtypeStruct((N, 1), jnp.float32),
            jax.ShapeDtypeStruct((N, 1), jnp.float32),
        ],
    )(beta.reshape(1), x)


_SC_CORES = 2
_SC_SUB = 16
_SC_W = _SC_CORES * _SC_SUB
_EPW = E // _SC_W
_C = 80
_NCH = _EPW // _C
_G = _C // 16


def _sc_edges(xn, xnb, norm1d, src, dst):
    mesh = plsc.VectorSubcoreMesh(core_axis_name="c", subcore_axis_name="s")

    buf = lambda: [
        pltpu.VMEM((_C,), jnp.int32),
        pltpu.VMEM((_C,), jnp.int32),
        pltpu.VMEM((_C, D), jnp.float32),
        pltpu.VMEM((_C, D), jnp.float32),
        pltpu.VMEM((_C,), jnp.float32),
        pltpu.VMEM((_C,), jnp.float32),
        pltpu.SemaphoreType.DMA,
        pltpu.SemaphoreType.DMA,
        pltpu.SemaphoreType.DMA,
    ]

    @functools.partial(
        pl.kernel,
        mesh=mesh,
        out_type=[
            jax.ShapeDtypeStruct((_SC_CORES, N, D), jnp.float32),
            jax.ShapeDtypeStruct((_SC_CORES, N), jnp.float32),
        ],
        scratch_types=buf() + buf() + [
            pltpu.VMEM_SHARED((N, D), jnp.float32),
            pltpu.VMEM_SHARED((N,), jnp.float32),
        ],
    )
    def k(xn_hbm, xnb_hbm, norm_hbm, src_hbm, dst_hbm, hout_hbm, aout_hbm,
          *refs):
        bufa, bufb = refs[0:9], refs[9:18]
        hsh, ash = refs[18], refs[19]
        cid = lax.axis_index("c")
        sid = lax.axis_index("s")
        wid = cid * _SC_SUB + sid

        _, _, srows0, _, _, we0 = bufa[0:6]

        z16 = jnp.zeros((16,), jnp.float32)

        def zrow(r, _):
            for cg in range(D // 16):
                srows0[r, pl.ds(cg * 16, 16)] = z16
            return 0

        lax.fori_loop(0, _C, zrow, 0)
        for g in range(_G):
            we0[pl.ds(g * 16, 16)] = z16

        def zh(j, _):
            pltpu.sync_copy(srows0, hsh.at[pl.ds(j * _C, _C)])
            return 0

        def za(j, _):
            pltpu.sync_copy(we0, ash.at[pl.ds(j * _C, _C)])
            return 0

        nblk = N // _C
        per = nblk // _SC_SUB + 1
        lo = sid * per
        hi = jnp.minimum(lo + per, nblk)
        lax.fori_loop(lo, hi, zh, 0)
        lax.fori_loop(lo, hi, za, 0)

        plsc.subcore_barrier()

        base = wid * _EPW
        lanes = lax.iota(jnp.int32, 16)

        def start(j, b):
            sidx, didx, srows, drows, nrm_v, we_v = b[0:6]
            g1, g2, g3 = b[6:9]
            off = base + j * _C
            pltpu.sync_copy(src_hbm.at[pl.ds(off, _C)], sidx)
            pltpu.sync_copy(dst_hbm.at[pl.ds(off, _C)], didx)
            pltpu.make_async_copy(xn_hbm.at[sidx], srows, g1).start()
            pltpu.make_async_copy(xnb_hbm.at[didx], drows, g2).start()
            pltpu.make_async_copy(norm_hbm.at[sidx], nrm_v, g3).start()

        def compute(b):
            sidx, didx, srows, drows, nrm_v, we_v = b[0:6]
            g1, g2, g3 = b[6:9]
            pltpu.make_async_copy(xn_hbm.at[sidx], srows, g1).wait()
            pltpu.make_async_copy(xnb_hbm.at[didx], drows, g2).wait()
            pltpu.make_async_copy(norm_hbm.at[sidx], nrm_v, g3).wait()

            def group(g, _):
                res = jnp.zeros((16,), jnp.float32)
                for el in range(16):
                    row = g * 16 + el
                    acc = jnp.zeros((16,), jnp.float32)
                    for cg in range(D // 16):
                        acc = acc + (srows[row, pl.ds(cg * 16, 16)]
                                     * drows[row, pl.ds(cg * 16, 16)])
                    for sh in (8, 4, 2, 1):
                        acc = acc + acc.at[lanes ^ sh].get(
                            mode="promise_in_bounds")
                    res = jnp.where(lanes == el, acc, res)
                we16 = jnp.exp(res)
                we_v[pl.ds(g * 16, 16)] = we16
                wn16 = we16 * nrm_v[pl.ds(g * 16, 16)]
                for el in range(16):
                    row = g * 16 + el
                    wb = wn16.at[jnp.full((16,), el, jnp.int32)].get(
                        mode="promise_in_bounds")
                    for cg in range(D // 16):
                        srows[row, pl.ds(cg * 16, 16)] = (
                            srows[row, pl.ds(cg * 16, 16)] * wb)
                return 0

            lax.fori_loop(0, _G, group, 0)

            pltpu.sync_copy(we_v, ash.at[didx], add=True)
            pltpu.sync_copy(srows, hsh.at[didx], add=True)

        start(0, bufa)

        def pair(p, _):
            start(2 * p + 1, bufb)
            compute(bufa)
            start(2 * p + 2, bufa)
            compute(bufb)
            return 0

        lax.fori_loop(0, (_NCH - 1) // 2, pair, 0)
        compute(bufa)

        plsc.subcore_barrier()

        @pl.when(sid == 0)
        def _():
            pltpu.sync_copy(hsh, hout_hbm.at[cid])
            pltpu.sync_copy(ash, aout_hbm.at[cid])

    return k(xn, xnb, norm1d, src, dst)


_RB2 = 400


def _combine_body(hn_ref, asum_ref, selfnum_ref, selfw_ref, wqkvT_ref,
                  bqkv_ref, h_ref, qkv_ref, bq_ref, kmax_ref, kaug_ref,
                  vaug_ref):
    num = hn_ref[0] + hn_ref[1] + selfnum_ref[...]
    den = asum_ref[0] + asum_ref[1] + selfw_ref[...]
    h = num / den
    h_ref[...] = h
    qkv = (jnp.dot(h, wqkvT_ref[...], preferred_element_type=jnp.float32)
           + bqkv_ref[...])
    qkv_ref[...] = qkv
    onecol = jnp.ones((qkv.shape[0], 1), jnp.float32)
    kaug_ref[...] = jnp.stack(
        [jnp.concatenate(
            [qkv[:, D + hd * HDIM:D + (hd + 1) * HDIM], onecol], axis=1)
         for hd in range(NHEAD)]).astype(jnp.bfloat16)
    vaug_ref[...] = jnp.stack(
        [jnp.concatenate(
            [qkv[:, 2 * D + hd * HDIM:2 * D + (hd + 1) * HDIM], onecol],
            axis=1)
         for hd in range(NHEAD)]).astype(jnp.bfloat16)
    scale = 1.0 / (HDIM ** 0.5)
    qn = [jnp.sqrt(jnp.sum(
        qkv[:, hd * HDIM:(hd + 1) * HDIM] ** 2, axis=1, keepdims=True))
        for hd in range(NHEAD)]
    bq_ref[...] = jnp.concatenate(qn, axis=1) * scale
    kn = [jnp.sqrt(jnp.max(jnp.sum(
        qkv[:, D + hd * HDIM:D + (hd + 1) * HDIM] ** 2, axis=1,
        keepdims=True))) for hd in range(NHEAD)]
    blockmax = jnp.stack(kn).reshape(1, NHEAD)

    @pl.when(pl.program_id(0) == 0)
    def _():
        kmax_ref[...] = blockmax

    @pl.when(pl.program_id(0) > 0)
    def _():
        kmax_ref[...] = jnp.maximum(kmax_ref[...], blockmax)


def _combine(hnum, asum, selfnum, selfw, WqkvT, bqkv2d):
    grid = N // _RB2
    return pl.pallas_call(
        _combine_body,
        grid=grid,
        in_specs=[
            pl.BlockSpec((2, _RB2, D), lambda i: (0, i, 0)),
            pl.BlockSpec((2, _RB2, 1), lambda i: (0, i, 0)),
            pl.BlockSpec((_RB2, D), lambda i: (i, 0)),
            pl.BlockSpec((_RB2, 1), lambda i: (i, 0)),
            pl.BlockSpec((D, 3 * D), lambda i: (0, 0)),
            pl.BlockSpec((1, 3 * D), lambda i: (0, 0)),
        ],
        out_specs=[
            pl.BlockSpec((_RB2, D), lambda i: (i, 0)),
            pl.BlockSpec((_RB2, 3 * D), lambda i: (i, 0)),
            pl.BlockSpec((_RB2, NHEAD), lambda i: (i, 0)),
            pl.BlockSpec((1, NHEAD), lambda i: (0, 0)),
            pl.BlockSpec((NHEAD, _RB2, HDIM + 1), lambda i: (0, i, 0)),
            pl.BlockSpec((NHEAD, _RB2, HDIM + 1), lambda i: (0, i, 0)),
        ],
        out_shape=[
            jax.ShapeDtypeStruct((N, D), jnp.float32),
            jax.ShapeDtypeStruct((N, 3 * D), jnp.float32),
            jax.ShapeDtypeStruct((N, NHEAD), jnp.float32),
            jax.ShapeDtypeStruct((1, NHEAD), jnp.float32),
            jax.ShapeDtypeStruct((NHEAD, N, HDIM + 1), jnp.bfloat16),
            jax.ShapeDtypeStruct((NHEAD, N, HDIM + 1), jnp.bfloat16),
        ],
    )(hnum, asum, selfnum, selfw, WqkvT, bqkv2d)


_RBQ = 400
_KC = 2000
_NKC = N // _KC


def _layernorm(x, g, b, eps=1e-5):
    m = jnp.mean(x, axis=-1, keepdims=True)
    v = jnp.mean((x - m) ** 2, axis=-1, keepdims=True)
    return (x - m) * jax.lax.rsqrt(v + eps) * g + b


def _dense_body(q_ref, kaug_ref, vaug_ref, h_ref, bq_ref, kmax_ref, woT, bo,
                ln1g, ln1b, w1T, b1, w2T, b2, ln2g, ln2b, fcT, fcb, outT,
                outb, o_ref):
    bf = jnp.bfloat16
    scale = 1.0 / (HDIM ** 0.5)
    ctx_parts = []
    for hd in range(NHEAD):
        sl = slice(hd * HDIM, (hd + 1) * HDIM)
        b = bq_ref[:, hd:hd + 1] * kmax_ref[0, hd]
        q = jnp.concatenate([q_ref[:, sl] * scale, -b], axis=1).astype(bf)
        acc = jnp.zeros((_RBQ, HDIM + 1), jnp.float32)
        for kc in range(_NKC):
            kblk = kaug_ref[hd, kc * _KC:(kc + 1) * _KC, :]
            vblk = vaug_ref[hd, kc * _KC:(kc + 1) * _KC, :]
            s = jax.lax.dot_general(
                q, kblk, (((1,), (1,)), ((), ())),
                preferred_element_type=jnp.float32)
            p = jnp.exp(s).astype(bf)
            acc = acc + jnp.dot(p, vblk, preferred_element_type=jnp.float32)
        ctx_parts.append(acc[:, :HDIM] / acc[:, HDIM:HDIM + 1])
    ctx = jnp.concatenate(ctx_parts, axis=1)

    attn = jnp.dot(ctx.astype(bf), woT[...].astype(bf),
                   preferred_element_type=jnp.float32) + bo[...]
    h1 = _layernorm(h_ref[...] + attn, ln1g[...], ln1b[...])
    ff = jnp.maximum(
        jnp.dot(h1.astype(bf), w1T[...].astype(bf),
                preferred_element_type=jnp.float32) + b1[...], 0.0)
    ff = jnp.dot(ff.astype(bf), w2T[...].astype(bf),
                 preferred_element_type=jnp.float32) + b2[...]
    h2 = _layernorm(h1 + ff, ln2g[...], ln2b[...])
    g = jnp.maximum(
        jnp.dot(h2.astype(bf), fcT[...].astype(bf),
                preferred_element_type=jnp.float32) + fcb[...], 0.0)
    o_ref[...] = (
        jnp.dot(g.astype(bf), outT[...].astype(bf),
                preferred_element_type=jnp.float32) + outb[...])


def _dense(q, kaug, vaug, h, bq, kmax, woT, bo, ln1g, ln1b, w1T, b1, w2T, b2,
           ln2g, ln2b, fcT, fcb, outT, outb):
    grid = N // _RBQ
    full = lambda r, c: pl.BlockSpec((r, c), lambda i: (0, 0))
    return pl.pallas_call(
        _dense_body,
        grid=grid,
        in_specs=[
            pl.BlockSpec((_RBQ, D), lambda i: (i, 0)),
            pl.BlockSpec((NHEAD, N, HDIM + 1), lambda i: (0, 0, 0)),
            pl.BlockSpec((NHEAD, N, HDIM + 1), lambda i: (0, 0, 0)),
            pl.BlockSpec((_RBQ, D), lambda i: (i, 0)),
            pl.BlockSpec((_RBQ, NHEAD), lambda i: (i, 0)),
            pl.BlockSpec(memory_space=pltpu.SMEM),
            full(D, D), full(1, D), full(1, D), full(1, D),
            full(D, DFF), full(1, DFF), full(DFF, D), full(1, D),
            full(1, D), full(1, D),
            full(D, HID), full(1, HID), full(HID, 2), full(1, 2),
        ],
        out_specs=[pl.BlockSpec((_RBQ, 2), lambda i: (i, 0))],
        out_shape=[jax.ShapeDtypeStruct((N, 2), jnp.float32)],
    )(q, kaug, vaug, h, bq, kmax, woT, bo, ln1g, ln1b, w1T, b1, w2T, b2,
      ln2g, ln2b, fcT, fcb, outT, outb)[0]


def kernel(x, edge_index, beta, Wqkv, bqkv, Wo, bo, ln1_g, ln1_b, W1, b1,
           W2, b2, ln2_g, ln2_b, fc_W, fc_b, out_W, out_b):
    xn, xnb, selfnum, selfw, nrm = _prep(x, beta)
    hnum, asum = _sc_edges(xn, xnb, nrm.reshape(N), edge_index[0],
                           edge_index[1])
    h, qkv, bq, kmax, kaug, vaug = _combine(
        hnum, asum.reshape(2, N, 1), selfnum, selfw, Wqkv.T,
        bqkv.reshape(1, 3 * D))
    q = qkv[:, :D]
    r2 = lambda a: a.reshape(1, -1)
    return _dense(q, kaug, vaug, h, bq, kmax, Wo.T, r2(bo), r2(ln1_g),
                  r2(ln1_b), W1.T, r2(b1), W2.T, r2(b2), r2(ln2_g),
                  r2(ln2_b), fc_W.T, r2(fc_b), out_W.T, r2(out_b))


# speedup vs baseline: 1.0167x; 1.0167x over previous
import functools

import jax
import jax.numpy as jnp
from jax import lax
from jax.experimental import pallas as pl
from jax.experimental.pallas import tpu as pltpu
from jax.experimental.pallas import tpu_sc as plsc

N = 10000
D = 128
E = 320000
NHEAD = 2
HDIM = D // NHEAD
DFF = 2048
HID = 256

_RB1 = 400


def _prep_body(beta_ref, x_ref, xn_ref, xnb_ref, selfnum_ref, selfw_ref,
               norm_ref):
    x = x_ref[...]
    beta = beta_ref[0]
    nsq = jnp.sum(x * x, axis=1, keepdims=True)
    nrm = jnp.maximum(jnp.sqrt(nsq), 1e-12)
    xn = x / nrm
    xn_ref[...] = xn
    xnb_ref[...] = xn * beta
    sw = jnp.exp(beta * jnp.sum(xn * xn, axis=1, keepdims=True))
    selfnum_ref[...] = x * sw
    selfw_ref[...] = sw
    norm_ref[...] = nrm


def _prep(x, beta):
    grid = N // _RB1
    return pl.pallas_call(
        _prep_body,
        grid=grid,
        in_specs=[
            pl.BlockSpec(memory_space=pltpu.SMEM),
            pl.BlockSpec((_RB1, D), lambda i: (i, 0)),
        ],
        out_specs=[
            pl.BlockSpec((_RB1, D), lambda i: (i, 0)),
            pl.BlockSpec((_RB1, D), lambda i: (i, 0)),
            pl.BlockSpec((_RB1, D), lambda i: (i, 0)),
            pl.BlockSpec((_RB1, 1), lambda i: (i, 0)),
            pl.BlockSpec((_RB1, 1), lambda i: (i, 0)),
        ],
        out_shape=[
            jax.ShapeDtypeStruct((N, D), jnp.float32),
            jax.ShapeDtypeStruct((N, D), jnp.float32),
            jax.ShapeDtypeStruct((N, D), jnp.float32),
            jax.ShapeDtypeStruct((N, 1), jnp.float32),
            jax.ShapeDtypeStruct((N, 1), jnp.float32),
        ],
    )(beta.reshape(1), x)


_SC_CORES = 2
_SC_SUB = 16
_SC_W = _SC_CORES * _SC_SUB
_EPW = E // _SC_W
_C = 80
_NCH = _EPW // _C
_G = _C // 16


def _sc_edges(xn, xnb, norm1d, src, dst):
    mesh = plsc.VectorSubcoreMesh(core_axis_name="c", subcore_axis_name="s")

    buf = lambda: [
        pltpu.VMEM((_C,), jnp.int32),
        pltpu.VMEM((_C,), jnp.int32),
        pltpu.VMEM((_C, D), jnp.float32),
        pltpu.VMEM((_C, D), jnp.float32),
        pltpu.VMEM((_C,), jnp.float32),
        pltpu.VMEM((_C,), jnp.float32),
        pltpu.SemaphoreType.DMA,
        pltpu.SemaphoreType.DMA,
        pltpu.SemaphoreType.DMA,
    ]

    @functools.partial(
        pl.kernel,
        mesh=mesh,
        out_type=[
            jax.ShapeDtypeStruct((_SC_CORES, N, D), jnp.float32),
            jax.ShapeDtypeStruct((_SC_CORES, N), jnp.float32),
        ],
        scratch_types=buf() + buf() + [
            pltpu.VMEM_SHARED((N, D), jnp.float32),
            pltpu.VMEM_SHARED((N,), jnp.float32),
        ],
    )
    def k(xn_hbm, xnb_hbm, norm_hbm, src_hbm, dst_hbm, hout_hbm, aout_hbm,
          *refs):
        bufa, bufb = refs[0:9], refs[9:18]
        hsh, ash = refs[18], refs[19]
        cid = lax.axis_index("c")
        sid = lax.axis_index("s")
        wid = cid * _SC_SUB + sid

        _, _, srows0, _, _, we0 = bufa[0:6]

        z16 = jnp.zeros((16,), jnp.float32)

        def zrow(r, _):
            for cg in range(D // 16):
                srows0[r, pl.ds(cg * 16, 16)] = z16
            return 0

        lax.fori_loop(0, _C, zrow, 0)
        for g in range(_G):
            we0[pl.ds(g * 16, 16)] = z16

        def zh(j, _):
            pltpu.sync_copy(srows0, hsh.at[pl.ds(j * _C, _C)])
            return 0

        def za(j, _):
            pltpu.sync_copy(we0, ash.at[pl.ds(j * _C, _C)])
            return 0

        nblk = N // _C
        per = nblk // _SC_SUB + 1
        lo = sid * per
        hi = jnp.minimum(lo + per, nblk)
        lax.fori_loop(lo, hi, zh, 0)
        lax.fori_loop(lo, hi, za, 0)

        plsc.subcore_barrier()

        base = wid * _EPW
        lanes = lax.iota(jnp.int32, 16)

        def start(j, b):
            sidx, didx, srows, drows, nrm_v, we_v = b[0:6]
            g1, g2, g3 = b[6:9]
            off = base + j * _C
            pltpu.sync_copy(src_hbm.at[pl.ds(off, _C)], sidx)
            pltpu.sync_copy(dst_hbm.at[pl.ds(off, _C)], didx)
            pltpu.make_async_copy(xn_hbm.at[sidx], srows, g1).start()
            pltpu.make_async_copy(xnb_hbm.at[didx], drows, g2).start()
            pltpu.make_async_copy(norm_hbm.at[sidx], nrm_v, g3).start()

        def compute(b):
            sidx, didx, srows, drows, nrm_v, we_v = b[0:6]
            g1, g2, g3 = b[6:9]
            pltpu.make_async_copy(xn_hbm.at[sidx], srows, g1).wait()
            pltpu.make_async_copy(xnb_hbm.at[didx], drows, g2).wait()
            pltpu.make_async_copy(norm_hbm.at[sidx], nrm_v, g3).wait()

            def group(g, _):
                res = jnp.zeros((16,), jnp.float32)
                for el in range(16):
                    row = g * 16 + el
                    acc = jnp.zeros((16,), jnp.float32)
                    for cg in range(D // 16):
                        acc = acc + (srows[row, pl.ds(cg * 16, 16)]
                                     * drows[row, pl.ds(cg * 16, 16)])
                    for sh in (8, 4, 2, 1):
                        acc = acc + acc.at[lanes ^ sh].get(
                            mode="promise_in_bounds")
                    res = jnp.where(lanes == el, acc, res)
                we16 = jnp.exp(res)
                we_v[pl.ds(g * 16, 16)] = we16
                wn16 = we16 * nrm_v[pl.ds(g * 16, 16)]
                for el in range(16):
                    row = g * 16 + el
                    wb = wn16.at[jnp.full((16,), el, jnp.int32)].get(
                        mode="promise_in_bounds")
                    for cg in range(D // 16):
                        srows[row, pl.ds(cg * 16, 16)] = (
                            srows[row, pl.ds(cg * 16, 16)] * wb)
                return 0

            lax.fori_loop(0, _G, group, 0)

            pltpu.sync_copy(we_v, ash.at[didx], add=True)
            pltpu.sync_copy(srows, hsh.at[didx], add=True)

        start(0, bufa)

        def pair(p, _):
            start(2 * p + 1, bufb)
            compute(bufa)
            start(2 * p + 2, bufa)
            compute(bufb)
            return 0

        lax.fori_loop(0, (_NCH - 1) // 2, pair, 0)
        compute(bufa)

        plsc.subcore_barrier()

        @pl.when(sid == 0)
        def _():
            pltpu.sync_copy(hsh, hout_hbm.at[cid])
            pltpu.sync_copy(ash, aout_hbm.at[cid])

    return k(xn, xnb, norm1d, src, dst)


_RB2 = 400


def _combine_body(hn_ref, asum_ref, selfnum_ref, selfw_ref, wqkvT_ref,
                  bqkv_ref, h_ref, qkv_ref, bq_ref, kmax_ref, kaug_ref,
                  vaug_ref):
    num = hn_ref[0] + hn_ref[1] + selfnum_ref[...]
    den = asum_ref[0] + asum_ref[1] + selfw_ref[...]
    h = num / den
    h_ref[...] = h
    qkv = (jnp.dot(h, wqkvT_ref[...], preferred_element_type=jnp.float32)
           + bqkv_ref[...])
    qkv_ref[...] = qkv
    onecol = jnp.ones((qkv.shape[0], 1), jnp.float32)
    kaug_ref[...] = jnp.stack(
        [jnp.concatenate(
            [qkv[:, D + hd * HDIM:D + (hd + 1) * HDIM], onecol], axis=1)
         for hd in range(NHEAD)]).astype(jnp.bfloat16)
    vaug_ref[...] = jnp.stack(
        [jnp.concatenate(
            [qkv[:, 2 * D + hd * HDIM:2 * D + (hd + 1) * HDIM], onecol],
            axis=1)
         for hd in range(NHEAD)]).astype(jnp.bfloat16)
    scale = 1.0 / (HDIM ** 0.5)
    qn = [jnp.sqrt(jnp.sum(
        qkv[:, hd * HDIM:(hd + 1) * HDIM] ** 2, axis=1, keepdims=True))
        for hd in range(NHEAD)]
    bq_ref[...] = jnp.concatenate(qn, axis=1) * scale
    kn = [jnp.sqrt(jnp.max(jnp.sum(
        qkv[:, D + hd * HDIM:D + (hd + 1) * HDIM] ** 2, axis=1,
        keepdims=True))) for hd in range(NHEAD)]
    blockmax = jnp.stack(kn).reshape(1, NHEAD)

    @pl.when(pl.program_id(0) == 0)
    def _():
        kmax_ref[...] = blockmax

    @pl.when(pl.program_id(0) > 0)
    def _():
        kmax_ref[...] = jnp.maximum(kmax_ref[...], blockmax)


def _combine(hnum, asum, selfnum, selfw, WqkvT, bqkv2d):
    grid = N // _RB2
    return pl.pallas_call(
        _combine_body,
        grid=grid,
        in_specs=[
            pl.BlockSpec((2, _RB2, D), lambda i: (0, i, 0)),
            pl.BlockSpec((2, _RB2, 1), lambda i: (0, i, 0)),
            pl.BlockSpec((_RB2, D), lambda i: (i, 0)),
            pl.BlockSpec((_RB2, 1), lambda i: (i, 0)),
            pl.BlockSpec((D, 3 * D), lambda i: (0, 0)),
            pl.BlockSpec((1, 3 * D), lambda i: (0, 0)),
        ],
        out_specs=[
            pl.BlockSpec((_RB2, D), lambda i: (i, 0)),
            pl.BlockSpec((_RB2, 3 * D), lambda i: (i, 0)),
            pl.BlockSpec((_RB2, NHEAD), lambda i: (i, 0)),
            pl.BlockSpec((1, NHEAD), lambda i: (0, 0)),
            pl.BlockSpec((NHEAD, _RB2, HDIM + 1), lambda i: (0, i, 0)),
            pl.BlockSpec((NHEAD, _RB2, HDIM + 1), lambda i: (0, i, 0)),
        ],
        out_shape=[
            jax.ShapeDtypeStruct((N, D), jnp.float32),
            jax.ShapeDtypeStruct((N, 3 * D), jnp.float32),
            jax.ShapeDtypeStruct((N, NHEAD), jnp.float32),
            jax.ShapeDtypeStruct((1, NHEAD), jnp.float32),
            jax.ShapeDtypeStruct((NHEAD, N, HDIM + 1), jnp.bfloat16),
            jax.ShapeDtypeStruct((NHEAD, N, HDIM + 1), jnp.bfloat16),
        ],
    )(hnum, asum, selfnum, selfw, WqkvT, bqkv2d)


_RBQ = 1000
_KC = 2000
_NKC = N // _KC


def _layernorm(x, g, b, eps=1e-5):
    m = jnp.mean(x, axis=-1, keepdims=True)
    v = jnp.mean((x - m) ** 2, axis=-1, keepdims=True)
    return (x - m) * jax.lax.rsqrt(v + eps) * g + b


def _dense_body(q_ref, kaug_ref, vaug_ref, h_ref, bq_ref, kmax_ref, woT, bo,
                ln1g, ln1b, w1T, b1, w2T, b2, ln2g, ln2b, fcT, fcb, outT,
                outb, o_ref):
    bf = jnp.bfloat16
    scale = 1.0 / (HDIM ** 0.5)
    ctx_parts = []
    for hd in range(NHEAD):
        sl = slice(hd * HDIM, (hd + 1) * HDIM)
        b = bq_ref[:, hd:hd + 1] * kmax_ref[0, hd]
        q = jnp.concatenate([q_ref[:, sl] * scale, -b], axis=1).astype(bf)
        acc = jnp.zeros((_RBQ, HDIM + 1), jnp.float32)
        for kc in range(_NKC):
            kblk = kaug_ref[hd, kc * _KC:(kc + 1) * _KC, :]
            vblk = vaug_ref[hd, kc * _KC:(kc + 1) * _KC, :]
            s = jax.lax.dot_general(
                q, kblk, (((1,), (1,)), ((), ())),
                preferred_element_type=jnp.float32)
            p = jnp.exp(s).astype(bf)
            acc = acc + jnp.dot(p, vblk, preferred_element_type=jnp.float32)
        ctx_parts.append(acc[:, :HDIM] / acc[:, HDIM:HDIM + 1])
    ctx = jnp.concatenate(ctx_parts, axis=1)

    attn = jnp.dot(ctx.astype(bf), woT[...].astype(bf),
                   preferred_element_type=jnp.float32) + bo[...]
    h1 = _layernorm(h_ref[...] + attn, ln1g[...], ln1b[...])
    ff = jnp.maximum(
        jnp.dot(h1.astype(bf), w1T[...].astype(bf),
                preferred_element_type=jnp.float32) + b1[...], 0.0)
    ff = jnp.dot(ff.astype(bf), w2T[...].astype(bf),
                 preferred_element_type=jnp.float32) + b2[...]
    h2 = _layernorm(h1 + ff, ln2g[...], ln2b[...])
    g = jnp.maximum(
        jnp.dot(h2.astype(bf), fcT[...].astype(bf),
                preferred_element_type=jnp.float32) + fcb[...], 0.0)
    o_ref[...] = (
        jnp.dot(g.astype(bf), outT[...].astype(bf),
                preferred_element_type=jnp.float32) + outb[...])


def _dense(q, kaug, vaug, h, bq, kmax, woT, bo, ln1g, ln1b, w1T, b1, w2T, b2,
           ln2g, ln2b, fcT, fcb, outT, outb):
    grid = N // _RBQ
    full = lambda r, c: pl.BlockSpec((r, c), lambda i: (0, 0))
    return pl.pallas_call(
        _dense_body,
        grid=grid,
        in_specs=[
            pl.BlockSpec((_RBQ, D), lambda i: (i, 0)),
            pl.BlockSpec((NHEAD, N, HDIM + 1), lambda i: (0, 0, 0)),
            pl.BlockSpec((NHEAD, N, HDIM + 1), lambda i: (0, 0, 0)),
            pl.BlockSpec((_RBQ, D), lambda i: (i, 0)),
            pl.BlockSpec((_RBQ, NHEAD), lambda i: (i, 0)),
            pl.BlockSpec(memory_space=pltpu.SMEM),
            full(D, D), full(1, D), full(1, D), full(1, D),
            full(D, DFF), full(1, DFF), full(DFF, D), full(1, D),
            full(1, D), full(1, D),
            full(D, HID), full(1, HID), full(HID, 2), full(1, 2),
        ],
        out_specs=[pl.BlockSpec((_RBQ, 2), lambda i: (i, 0))],
        out_shape=[jax.ShapeDtypeStruct((N, 2), jnp.float32)],
    )(q, kaug, vaug, h, bq, kmax, woT, bo, ln1g, ln1b, w1T, b1, w2T, b2,
      ln2g, ln2b, fcT, fcb, outT, outb)[0]


def kernel(x, edge_index, beta, Wqkv, bqkv, Wo, bo, ln1_g, ln1_b, W1, b1,
           W2, b2, ln2_g, ln2_b, fc_W, fc_b, out_W, out_b):
    xn, xnb, selfnum, selfw, nrm = _prep(x, beta)
    hnum, asum = _sc_edges(xn, xnb, nrm.reshape(N), edge_index[0],
                           edge_index[1])
    h, qkv, bq, kmax, kaug, vaug = _combine(
        hnum, asum.reshape(2, N, 1), selfnum, selfw, Wqkv.T,
        bqkv.reshape(1, 3 * D))
    q = qkv[:, :D]
    r2 = lambda a: a.reshape(1, -1)
    return _dense(q, kaug, vaug, h, bq, kmax, Wo.T, r2(bo), r2(ln1_g),
                  r2(ln1_b), W1.T, r2(b1), W2.T, r2(b2), r2(ln2_g),
                  r2(ln2_b), fc_W.T, r2(fc_b), out_W.T, r2(out_b))
